# baseline (speedup 1.0000x reference)
import functools

import jax
import jax.numpy as jnp
from jax import lax
from jax.experimental import pallas as pl
from jax.experimental.pallas import tpu as pltpu
from jax.experimental.pallas import tpu_sc as plsc

NUM_EMB = 1024
DIM = 64
BATCH = 16
PIX = 32 * 32
N = BATCH * PIX
_LOSS_SCALE = 1.25 / (N * DIM)

_NC, _NS = 2, 16
_NW = _NC * _NS
_ROWS_PER_W = N // _NW
_CHUNK = 128
_NCHUNK = _ROWS_PER_W // _CHUNK


def _dist_body(x_ref, w_ref, idx_ref, loss_ref):
    b = pl.program_id(0)
    X = x_ref[0].reshape(DIM, PIX)
    Wt = w_ref[...]
    Wm2 = Wt * (-2.0)
    mm2 = lax.dot_general(Wm2, X, dimension_numbers=(((1,), (0,)), ((), ())),
                          preferred_element_type=jnp.float32)
    wn = jnp.sum(Wt * Wt, axis=1, keepdims=True)
    xn = jnp.sum(X * X, axis=0, keepdims=True)
    D = (xn + wn) + mm2
    minv = jnp.min(D, axis=0, keepdims=True)
    codes = lax.broadcasted_iota(jnp.int32, D.shape, 0)
    idx = jnp.min(jnp.where(D == minv, codes, NUM_EMB),
                  axis=0, keepdims=True)
    idx_ref[0] = idx.reshape(PIX // 128, 128)
    partial = jnp.sum(minv).reshape(1, 1)
    prev = jnp.where(b == 0, jnp.zeros((1, 1), jnp.float32), loss_ref[...])
    tot = prev + partial
    loss_ref[...] = jnp.where(b == BATCH - 1, tot * _LOSS_SCALE, tot)


def _distances_argmin(x, weight):
    idx3, loss = pl.pallas_call(
        _dist_body,
        grid=(BATCH,),
        in_specs=[
            pl.BlockSpec((1, DIM, 32, 32), lambda b: (b, 0, 0, 0)),
            pl.BlockSpec((NUM_EMB, DIM), lambda b: (0, 0)),
        ],
        out_specs=[
            pl.BlockSpec((1, PIX // 128, 128), lambda b: (b, 0, 0)),
            pl.BlockSpec((1, 1), lambda b: (0, 0)),
        ],
        out_shape=[
            jax.ShapeDtypeStruct((BATCH, PIX // 128, 128), jnp.int32),
            jax.ShapeDtypeStruct((1, 1), jnp.float32),
        ],
    )(x, weight)
    return idx3, loss.reshape(())


@functools.lru_cache(maxsize=1)
def _make_gather():
    mesh = plsc.VectorSubcoreMesh(core_axis_name="c", subcore_axis_name="s")
    half = PIX // 128 // 2

    @functools.partial(
        pl.kernel, mesh=mesh,
        compiler_params=pltpu.CompilerParams(use_tc_tiling_on_sc=False),
        out_type=jax.ShapeDtypeStruct((N, DIM), jnp.float32),
        scratch_types=[
            pltpu.VMEM((_NCHUNK, _CHUNK), jnp.int32),
            pltpu.VMEM((_ROWS_PER_W, DIM), jnp.float32),
            pltpu.SemaphoreType.DMA,
        ],
    )
    def gather(table_hbm, idx_hbm, out_hbm, idx_v, rows_v, sem):
        wid = lax.axis_index("s") * _NC + lax.axis_index("c")
        b = wid // 2
        h = wid % 2
        pltpu.sync_copy(idx_hbm.at[b, pl.ds(h * half, half)], idx_v)
        copies = []
        for j in range(_NCHUNK):
            copies.append(pltpu.async_copy(
                table_hbm.at[idx_v.at[j]],
                rows_v.at[pl.ds(j * _CHUNK, _CHUNK)], sem))
        for c in copies:
            c.wait()
        pltpu.sync_copy(rows_v, out_hbm.at[pl.ds(wid * _ROWS_PER_W,
                                                 _ROWS_PER_W)])

    return gather


def kernel(inputs, weight):
    idx, loss = _distances_argmin(inputs, weight)
    quantized = _make_gather()(weight, idx)
    out = quantized.reshape(BATCH, 32, 32, DIM).transpose(0, 3, 1, 2)
    return (out, loss)


# speedup vs baseline: 1.1825x; 1.1825x over previous
import functools

import jax
import jax.numpy as jnp
from jax import lax
from jax.experimental import pallas as pl
from jax.experimental.pallas import tpu as pltpu
from jax.experimental.pallas import tpu_sc as plsc

NUM_EMB = 1024
DIM = 64
BATCH = 16
PIX = 32 * 32
N = BATCH * PIX
_LOSS_SCALE = 1.25 / (N * DIM)

_NC, _NS = 2, 16
_NW = _NC * _NS
_ROWS_PER_W = N // _NW
_CHUNK = 128
_NCHUNK = _ROWS_PER_W // _CHUNK


def _dist_body(x_ref, w_ref, idx_ref, loss_ref):
    b = pl.program_id(0)
    X = x_ref[0]
    Wt = w_ref[...]
    Wm2 = Wt * (-2.0)
    mm2 = lax.dot_general(Wm2, X, dimension_numbers=(((1,), (0,)), ((), ())),
                          preferred_element_type=jnp.float32)
    wn = jnp.sum(Wt * Wt, axis=1, keepdims=True)
    xn = jnp.sum(X * X, axis=0, keepdims=True)
    D = (xn + wn) + mm2
    minv = jnp.min(D, axis=0, keepdims=True)
    codes = lax.broadcasted_iota(jnp.int32, D.shape, 0)
    idx = jnp.min(jnp.where(D == minv, codes, NUM_EMB),
                  axis=0, keepdims=True)
    idx_ref[0] = idx.reshape(PIX // 128, 128)
    partial = jnp.sum(minv).reshape(1, 1)
    prev = jnp.where(b == 0, jnp.zeros((1, 1), jnp.float32), loss_ref[...])
    tot = prev + partial
    loss_ref[...] = jnp.where(b == BATCH - 1, tot * _LOSS_SCALE, tot)


def _distances_argmin(x, weight):
    idx3, loss = pl.pallas_call(
        _dist_body,
        grid=(BATCH,),
        in_specs=[
            pl.BlockSpec((1, DIM, PIX), lambda b: (b, 0, 0)),
            pl.BlockSpec((NUM_EMB, DIM), lambda b: (0, 0)),
        ],
        out_specs=[
            pl.BlockSpec((1, PIX // 128, 128), lambda b: (b, 0, 0)),
            pl.BlockSpec((1, 1), lambda b: (0, 0)),
        ],
        out_shape=[
            jax.ShapeDtypeStruct((BATCH, PIX // 128, 128), jnp.int32),
            jax.ShapeDtypeStruct((1, 1), jnp.float32),
        ],
    )(x, weight)
    return idx3, loss.reshape(())


@functools.lru_cache(maxsize=1)
def _make_gather():
    mesh = plsc.VectorSubcoreMesh(core_axis_name="c", subcore_axis_name="s")
    half = PIX // 128 // 2

    @functools.partial(
        pl.kernel, mesh=mesh,
        compiler_params=pltpu.CompilerParams(use_tc_tiling_on_sc=False),
        out_type=jax.ShapeDtypeStruct((N, DIM), jnp.float32),
        scratch_types=[
            pltpu.VMEM((_NCHUNK, _CHUNK), jnp.int32),
            pltpu.VMEM((_ROWS_PER_W, DIM), jnp.float32),
            pltpu.SemaphoreType.DMA,
        ],
    )
    def gather(table_hbm, idx_hbm, out_hbm, idx_v, rows_v, sem):
        wid = lax.axis_index("s") * _NC + lax.axis_index("c")
        b = wid // 2
        h = wid % 2
        pltpu.sync_copy(idx_hbm.at[b, pl.ds(h * half, half)], idx_v)
        copies = []
        for j in range(_NCHUNK):
            copies.append(pltpu.async_copy(
                table_hbm.at[idx_v.at[j]],
                rows_v.at[pl.ds(j * _CHUNK, _CHUNK)], sem))
        for c in copies:
            c.wait()
        pltpu.sync_copy(rows_v, out_hbm.at[pl.ds(wid * _ROWS_PER_W,
                                                 _ROWS_PER_W)])

    return gather


def kernel(inputs, weight):
    x = inputs.reshape(BATCH, DIM, PIX)
    idx, loss = _distances_argmin(x, weight)
    quantized = _make_gather()(weight, idx)
    out = quantized.reshape(BATCH, 32, 32, DIM).transpose(0, 3, 1, 2)
    return (out, loss)


# speedup vs baseline: 1.6241x; 1.3735x over previous
import functools

import jax
import jax.numpy as jnp
from jax import lax
from jax.experimental import pallas as pl
from jax.experimental.pallas import tpu as pltpu
from jax.experimental.pallas import tpu_sc as plsc

NUM_EMB = 1024
DIM = 64
BATCH = 16
PIX = 32 * 32
N = BATCH * PIX
_LOSS_SCALE = 1.25 / (N * DIM)

_NC, _NS = 2, 16
_NW = _NC * _NS
_ROWS_PER_W = N // _NW
_CHUNK = 128
_NCHUNK = _ROWS_PER_W // _CHUNK


def _dist_body(x_ref, w_ref, idx_ref, loss_ref):
    b = pl.program_id(0)
    X = x_ref[0]
    Wt = w_ref[...]
    Wm2 = Wt * (-2.0)
    mm2 = lax.dot_general(Wm2, X, dimension_numbers=(((1,), (0,)), ((), ())),
                          preferred_element_type=jnp.float32)
    wn = jnp.sum(Wt * Wt, axis=1, keepdims=True)
    xn = jnp.sum(X * X, axis=0, keepdims=True)
    D = (xn + wn) + mm2
    minv = jnp.min(D, axis=0, keepdims=True)
    codes = lax.broadcasted_iota(jnp.int32, D.shape, 0)
    idx = jnp.min(jnp.where(D == minv, codes, NUM_EMB),
                  axis=0, keepdims=True)
    E = jnp.where(codes == idx, 1.0, 0.0)
    idx_ref[0] = lax.dot_general(
        Wt, E, dimension_numbers=(((0,), (0,)), ((), ())),
        preferred_element_type=jnp.float32)
    partial = jnp.sum(minv).reshape(1, 1)
    prev = jnp.where(b == 0, jnp.zeros((1, 1), jnp.float32), loss_ref[...])
    tot = prev + partial
    loss_ref[...] = jnp.where(b == BATCH - 1, tot * _LOSS_SCALE, tot)


def _distances_argmin(x, weight):
    idx3, loss = pl.pallas_call(
        _dist_body,
        grid=(BATCH,),
        in_specs=[
            pl.BlockSpec((1, DIM, PIX), lambda b: (b, 0, 0)),
            pl.BlockSpec((NUM_EMB, DIM), lambda b: (0, 0)),
        ],
        out_specs=[
            pl.BlockSpec((1, DIM, PIX), lambda b: (b, 0, 0)),
            pl.BlockSpec((1, 1), lambda b: (0, 0)),
        ],
        out_shape=[
            jax.ShapeDtypeStruct((BATCH, DIM, PIX), jnp.float32),
            jax.ShapeDtypeStruct((1, 1), jnp.float32),
        ],
    )(x, weight)
    return idx3, loss.reshape(())


@functools.lru_cache(maxsize=1)
def _make_gather():
    mesh = plsc.VectorSubcoreMesh(core_axis_name="c", subcore_axis_name="s")
    half = PIX // 128 // 2

    @functools.partial(
        pl.kernel, mesh=mesh,
        compiler_params=pltpu.CompilerParams(use_tc_tiling_on_sc=False),
        out_type=jax.ShapeDtypeStruct((N, DIM), jnp.float32),
        scratch_types=[
            pltpu.VMEM((_NCHUNK, _CHUNK), jnp.int32),
            pltpu.VMEM((_ROWS_PER_W, DIM), jnp.float32),
            pltpu.SemaphoreType.DMA,
        ],
    )
    def gather(table_hbm, idx_hbm, out_hbm, idx_v, rows_v, sem):
        wid = lax.axis_index("s") * _NC + lax.axis_index("c")
        b = wid // 2
        h = wid % 2
        pltpu.sync_copy(idx_hbm.at[b, pl.ds(h * half, half)], idx_v)
        copies = []
        for j in range(_NCHUNK):
            copies.append(pltpu.async_copy(
                table_hbm.at[idx_v.at[j]],
                rows_v.at[pl.ds(j * _CHUNK, _CHUNK)], sem))
        for c in copies:
            c.wait()
        pltpu.sync_copy(rows_v, out_hbm.at[pl.ds(wid * _ROWS_PER_W,
                                                 _ROWS_PER_W)])

    return gather


def kernel(inputs, weight):
    x = inputs.reshape(BATCH, DIM, PIX)
    qt, loss = _distances_argmin(x, weight)
    out = qt.reshape(BATCH, DIM, 32, 32)
    return (out, loss)
